# baseline (speedup 1.0000x reference)
import jax
import jax.numpy as jnp
from jax.experimental import pallas as pl
from jax.experimental.pallas import tpu as pltpu

_B = 2
_N = 512
_D = 128


def _soft_hd_kernel(wp_ref, out_ref):
    _W0 = 2 * _B * _N
    P1 = wp_ref[0:_B * _N, :]
    P2 = wp_ref[_B * _N:2 * _B * _N, :]
    W = wp_ref[_W0:_W0 + _D, :]
    Wd1 = wp_ref[_W0 + _D:_W0 + _D + 64, :]
    Wi1 = wp_ref[_W0 + _D + 64:_W0 + _D + 128, :]
    asrc = wp_ref[_W0 + 256:_W0 + 257, :]
    adst = wp_ref[_W0 + 264:_W0 + 265, :]
    bgat = wp_ref[_W0 + 272:_W0 + 273, :]
    bd1 = wp_ref[_W0 + 280:_W0 + 281, 0:64]
    bi1 = wp_ref[_W0 + 288:_W0 + 289, 0:64]
    w2d_row = wp_ref[_W0 + 296:_W0 + 297, 0:64]
    w2i_row = wp_ref[_W0 + 297:_W0 + 298, 0:64]
    srow = wp_ref[_W0 + 298:_W0 + 299, :]
    lane = jax.lax.broadcasted_iota(jnp.int32, (1, _D), 1)
    bd2 = jnp.sum(jnp.where(lane == 0, srow, 0.0))
    bi2 = jnp.sum(jnp.where(lane == 1, srow, 0.0))
    mean_del = jnp.sum(
        jnp.where((lane >= 2) & (lane < 7), srow, 0.0)) / 5.0
    mean_ins = jnp.sum(
        jnp.where((lane >= 7) & (lane < 12), srow, 0.0)) / 5.0

    M = _B * _N
    idx = jax.lax.broadcasted_iota(jnp.int32, (M, 1), 0)
    pos = jax.lax.bitwise_and(idx, _N - 1)
    has_m = (pos >= 1).astype(jnp.float32)
    has_p = (pos <= _N - 2).astype(jnp.float32)
    neg_big = jnp.float32(-1e30)

    att = jnp.concatenate(
        [jax.lax.transpose(asrc, (1, 0)),
         jax.lax.transpose(adst, (1, 0))], axis=1)

    def lrelu(x):
        return jnp.where(x >= 0, x, 0.2 * x)

    def gat(p):
        h = jnp.dot(p, W.T, preferred_element_type=jnp.float32)
        e = jnp.dot(h, att, preferred_element_type=jnp.float32)
        es = e[:, 0:1]
        ed = e[:, 1:2]
        es_m = jnp.roll(es, 1, axis=0)
        es_p = jnp.roll(es, -1, axis=0)
        e_m = jnp.where(has_m > 0, lrelu(es_m + ed), neg_big)
        e_s = lrelu(es + ed)
        e_p = jnp.where(has_p > 0, lrelu(es_p + ed), neg_big)
        m = jnp.maximum(jnp.maximum(e_m, e_p), e_s)
        w_m = jnp.exp(e_m - m) * has_m
        w_s = jnp.exp(e_s - m)
        w_p = jnp.exp(e_p - m) * has_p
        inv_s = 1.0 / (w_m + w_s + w_p)
        a_m = w_m * inv_s
        a_s = w_s * inv_s
        a_p = w_p * inv_s
        h_m = jnp.roll(h, 1, axis=0)
        h_p = jnp.roll(h, -1, axis=0)
        return a_m * h_m + a_s * h + a_p * h_p + bgat

    H1 = gat(P1)
    H2 = gat(P2)

    U = jnp.concatenate([-2.0 * P1, -2.0 * H1], axis=1)
    V = jnp.concatenate([P2, H2], axis=1)
    UU = U * U
    VV = V * V
    quarter = jnp.full((2 * _D, 8), 0.25, dtype=jnp.float32)
    aa_t = jnp.dot(UU, quarter,
                   preferred_element_type=jnp.float32)[:, 0:1]
    ones_row = jnp.full((1, 2 * _D), 1.0, dtype=jnp.float32)

    def mlp_q(p, W1, b1):
        return jnp.maximum(
            jnp.dot(p, W1.T, preferred_element_type=jnp.float32) + b1,
            0.0)

    q1 = mlp_q(P1, Wd1, bd1)
    q2 = mlp_q(P2, Wi1, bi1)
    w2d = jnp.concatenate(
        [jax.lax.transpose(w2d_row, (1, 0)),
         jnp.zeros((_D // 2, 7), jnp.float32)], axis=1)
    r1 = jnp.dot(q1, w2d, preferred_element_type=jnp.float32)[:, 0:1]
    d1 = mean_del + jnp.abs(r1 + bd2)
    r2 = jnp.dot(w2i_row, q2.T,
                 preferred_element_type=jnp.float32)
    d2 = mean_ins + jnp.abs(r2 + bi2)

    for b in range(_B):
        lo, hi = b * _N, (b + 1) * _N
        g2 = jnp.dot(U[lo:hi, :], V[lo:hi, :].T,
                     preferred_element_type=jnp.float32)
        bb = jnp.dot(ones_row, VV[lo:hi, :].T,
                     preferred_element_type=jnp.float32)
        y = (g2 + aa_t[lo:hi, :]) + bb
        colmin = jnp.maximum(jnp.min(y, axis=0, keepdims=True), 0.0)
        rowmin = jnp.maximum(jnp.min(y, axis=1, keepdims=True), 0.0)
        a_v = jnp.minimum(colmin, 2.0 * d2[:, lo:hi])
        b_v = jnp.minimum(rowmin, 2.0 * d1[lo:hi, :])
        out_ref[b] = (jnp.sum(a_v) + jnp.sum(b_v)) / jnp.float32(4 * _N)


def kernel(dense_t1, dense_t2, t1_mask, t2_mask, W_gat, att_src, att_dst,
           b_gat, Wd1, bd1, Wd2, bd2, Wi1, bi1, Wi2, bi2, emb_del, emb_ins):
    del t1_mask, t2_mask
    z7 = jnp.zeros((7 * _D,), jnp.float32)
    z64 = jnp.zeros((_D // 2,), jnp.float32)
    z116 = jnp.zeros((116,), jnp.float32)
    z5 = jnp.zeros((5 * _D,), jnp.float32)
    rows = 2 * _B * _N + 304
    packed = jnp.concatenate([
        dense_t1.ravel(), dense_t2.ravel(),
        W_gat.ravel(), Wd1.ravel(), Wi1.ravel(),
        att_src, z7, att_dst, z7, b_gat, z7,
        bd1, z64, z7, bi1, z64, z7,
        Wd2.ravel(), z64, Wi2.ravel(), z64,
        bd2, bi2, emb_del[_N - 5:_N, 0], emb_ins[_N - 5:_N, 0], z116, z5,
    ]).reshape(rows, _D)

    out = pl.pallas_call(
        _soft_hd_kernel,
        grid=(1,),
        in_specs=[pl.BlockSpec((rows, _D), lambda i: (0, 0))],
        out_specs=pl.BlockSpec(memory_space=pltpu.SMEM),
        out_shape=jax.ShapeDtypeStruct((_B,), jnp.float32),
    )(packed)
    return out


# speedup vs baseline: 3.7171x; 3.7171x over previous
import jax
import jax.numpy as jnp
from jax.experimental import pallas as pl
from jax.experimental.pallas import tpu as pltpu

_B = 2
_N = 512
_D = 128


def _soft_hd_kernel(wp_ref, out_ref):
    _W0 = 2 * _B * _N
    P1 = wp_ref[0:_B * _N, :]
    P2 = wp_ref[_B * _N:2 * _B * _N, :]
    W = wp_ref[_W0:_W0 + _D, :]
    Wd1 = wp_ref[_W0 + _D:_W0 + _D + 64, :]
    Wi1 = wp_ref[_W0 + _D + 64:_W0 + _D + 128, :]
    asrc = wp_ref[_W0 + 256:_W0 + 257, :]
    adst = wp_ref[_W0 + 264:_W0 + 265, :]
    bgat = wp_ref[_W0 + 272:_W0 + 273, :]
    bd1 = wp_ref[_W0 + 280:_W0 + 281, 0:64]
    bi1 = wp_ref[_W0 + 288:_W0 + 289, 0:64]
    w2d_row = wp_ref[_W0 + 296:_W0 + 297, 0:64]
    w2i_row = wp_ref[_W0 + 297:_W0 + 298, 0:64]
    srow = wp_ref[_W0 + 298:_W0 + 299, :]
    lane = jax.lax.broadcasted_iota(jnp.int32, (1, _D), 1)
    bd2 = jnp.sum(jnp.where(lane == 0, srow, 0.0))
    bi2 = jnp.sum(jnp.where(lane == 1, srow, 0.0))
    mean_del = jnp.sum(
        jnp.where((lane >= 2) & (lane < 7), srow, 0.0)) / 5.0
    mean_ins = jnp.sum(
        jnp.where((lane >= 7) & (lane < 12), srow, 0.0)) / 5.0

    M = _B * _N
    idx = jax.lax.broadcasted_iota(jnp.int32, (M, 1), 0)
    pos = jax.lax.bitwise_and(idx, _N - 1)
    has_m = (pos >= 1).astype(jnp.float32)
    has_p = (pos <= _N - 2).astype(jnp.float32)
    neg_big = jnp.float32(-1e30)

    att = jnp.concatenate(
        [jax.lax.transpose(asrc, (1, 0)),
         jax.lax.transpose(adst, (1, 0))], axis=1)

    def lrelu(x):
        return jnp.where(x >= 0, x, 0.2 * x)

    def gat(p):
        h = jnp.dot(p, W.T, preferred_element_type=jnp.float32)
        e = jnp.dot(h, att, preferred_element_type=jnp.float32)
        es = e[:, 0:1]
        ed = e[:, 1:2]
        es_m = jnp.roll(es, 1, axis=0)
        es_p = jnp.roll(es, -1, axis=0)
        e_m = jnp.where(has_m > 0, lrelu(es_m + ed), neg_big)
        e_s = lrelu(es + ed)
        e_p = jnp.where(has_p > 0, lrelu(es_p + ed), neg_big)
        m = jnp.maximum(jnp.maximum(e_m, e_p), e_s)
        w_m = jnp.exp(e_m - m) * has_m
        w_s = jnp.exp(e_s - m)
        w_p = jnp.exp(e_p - m) * has_p
        inv_s = 1.0 / (w_m + w_s + w_p)
        a_m = w_m * inv_s
        a_s = w_s * inv_s
        a_p = w_p * inv_s
        h_m = jnp.roll(h, 1, axis=0)
        h_p = jnp.roll(h, -1, axis=0)
        return a_m * h_m + a_s * h + a_p * h_p + bgat

    H1 = gat(P1)
    H2 = gat(P2)

    U = jnp.concatenate([-2.0 * P1, -2.0 * H1], axis=1)
    V = jnp.concatenate([P2, H2], axis=1)
    UU = U * U
    VV = V * V
    quarter = jnp.full((2 * _D, 8), 0.25, dtype=jnp.float32)
    aa_t = jnp.dot(UU, quarter,
                   preferred_element_type=jnp.float32)[:, 0:1]
    ones_row = jnp.full((1, 2 * _D), 1.0, dtype=jnp.float32)

    def mlp_q(p, W1, b1):
        return jnp.maximum(
            jnp.dot(p, W1.T, preferred_element_type=jnp.float32) + b1,
            0.0)

    q1 = mlp_q(P1, Wd1, bd1)
    q2 = mlp_q(P2, Wi1, bi1)
    w2d = jnp.concatenate(
        [jax.lax.transpose(w2d_row, (1, 0)),
         jnp.zeros((_D // 2, 7), jnp.float32)], axis=1)
    r1 = jnp.dot(q1, w2d, preferred_element_type=jnp.float32)[:, 0:1]
    d1 = mean_del + jnp.abs(r1 + bd2)
    r2 = jnp.dot(w2i_row, q2.T,
                 preferred_element_type=jnp.float32)
    d2 = mean_ins + jnp.abs(r2 + bi2)

    for b in range(_B):
        lo, hi = b * _N, (b + 1) * _N
        g2 = jnp.dot(U[lo:hi, :], V[lo:hi, :].T,
                     preferred_element_type=jnp.float32)
        bb = jnp.dot(ones_row, VV[lo:hi, :].T,
                     preferred_element_type=jnp.float32)
        y = (g2 + aa_t[lo:hi, :]) + bb
        colmin = jnp.maximum(jnp.min(y, axis=0, keepdims=True), 0.0)
        rowmin = jnp.maximum(jnp.min(y, axis=1, keepdims=True), 0.0)
        a_v = jnp.minimum(colmin, 2.0 * d2[:, lo:hi])
        b_v = jnp.minimum(rowmin, 2.0 * d1[lo:hi, :])
        out_ref[b] = (jnp.sum(a_v) + jnp.sum(b_v)) / jnp.float32(4 * _N)


def kernel(dense_t1, dense_t2, t1_mask, t2_mask, W_gat, att_src, att_dst,
           b_gat, Wd1, bd1, Wd2, bd2, Wi1, bi1, Wi2, bi2, emb_del, emb_ins):
    del t1_mask, t2_mask
    z7 = jnp.zeros((7 * _D,), jnp.float32)
    z64 = jnp.zeros((_D // 2,), jnp.float32)
    z116 = jnp.zeros((116,), jnp.float32)
    z5 = jnp.zeros((5 * _D,), jnp.float32)
    rows = 2 * _B * _N + 304
    small = jnp.concatenate([
        att_src, z7, att_dst, z7, b_gat, z7,
        bd1, z64, z7, bi1, z64, z7,
        Wd2.ravel(), z64, Wi2.ravel(), z64,
        bd2, bi2, emb_del[_N - 5:_N, 0], emb_ins[_N - 5:_N, 0], z116, z5,
    ]).reshape(48, _D)
    packed = jnp.concatenate([
        dense_t1.reshape(_B * _N, _D), dense_t2.reshape(_B * _N, _D),
        W_gat, Wd1, Wi1, small,
    ], axis=0)

    out = pl.pallas_call(
        _soft_hd_kernel,
        grid=(1,),
        in_specs=[pl.BlockSpec((rows, _D), lambda i: (0, 0))],
        out_specs=pl.BlockSpec(memory_space=pltpu.SMEM),
        out_shape=jax.ShapeDtypeStruct((_B,), jnp.float32),
    )(packed)
    return out


# speedup vs baseline: 4.0777x; 1.0970x over previous
import jax
import jax.numpy as jnp
from jax.experimental import pallas as pl
from jax.experimental.pallas import tpu as pltpu

_B = 2
_N = 512
_D = 128


def _soft_hd_kernel(p1_ref, p2_ref, wp_ref, out_ref):
    _W0 = 0
    P1 = p1_ref[...]
    P2 = p2_ref[...]
    W = wp_ref[_W0:_W0 + _D, :]
    Wd1 = wp_ref[_W0 + _D:_W0 + _D + 64, :]
    Wi1 = wp_ref[_W0 + _D + 64:_W0 + _D + 128, :]
    asrc = wp_ref[_W0 + 256:_W0 + 257, :]
    adst = wp_ref[_W0 + 264:_W0 + 265, :]
    bgat = wp_ref[_W0 + 272:_W0 + 273, :]
    bd1 = wp_ref[_W0 + 280:_W0 + 281, 0:64]
    bi1 = wp_ref[_W0 + 288:_W0 + 289, 0:64]
    w2d_row = wp_ref[_W0 + 296:_W0 + 297, 0:64]
    w2i_row = wp_ref[_W0 + 297:_W0 + 298, 0:64]
    srow = wp_ref[_W0 + 298:_W0 + 299, :]
    lane = jax.lax.broadcasted_iota(jnp.int32, (1, _D), 1)
    bd2 = jnp.sum(jnp.where(lane == 0, srow, 0.0))
    bi2 = jnp.sum(jnp.where(lane == 1, srow, 0.0))
    mean_del = jnp.sum(
        jnp.where((lane >= 2) & (lane < 7), srow, 0.0)) / 5.0
    mean_ins = jnp.sum(
        jnp.where((lane >= 7) & (lane < 12), srow, 0.0)) / 5.0

    M = _B * _N
    idx = jax.lax.broadcasted_iota(jnp.int32, (M, 1), 0)
    pos = jax.lax.bitwise_and(idx, _N - 1)
    has_m = (pos >= 1).astype(jnp.float32)
    has_p = (pos <= _N - 2).astype(jnp.float32)
    neg_big = jnp.float32(-1e30)

    att = jnp.concatenate(
        [jax.lax.transpose(asrc, (1, 0)),
         jax.lax.transpose(adst, (1, 0))], axis=1)

    def lrelu(x):
        return jnp.where(x >= 0, x, 0.2 * x)

    def gat(p):
        h = jnp.dot(p, W.T, preferred_element_type=jnp.float32)
        e = jnp.dot(h, att, preferred_element_type=jnp.float32)
        es = e[:, 0:1]
        ed = e[:, 1:2]
        es_m = jnp.roll(es, 1, axis=0)
        es_p = jnp.roll(es, -1, axis=0)
        e_m = jnp.where(has_m > 0, lrelu(es_m + ed), neg_big)
        e_s = lrelu(es + ed)
        e_p = jnp.where(has_p > 0, lrelu(es_p + ed), neg_big)
        m = jnp.maximum(jnp.maximum(e_m, e_p), e_s)
        w_m = jnp.exp(e_m - m) * has_m
        w_s = jnp.exp(e_s - m)
        w_p = jnp.exp(e_p - m) * has_p
        inv_s = 1.0 / (w_m + w_s + w_p)
        a_m = w_m * inv_s
        a_s = w_s * inv_s
        a_p = w_p * inv_s
        h_m = jnp.roll(h, 1, axis=0)
        h_p = jnp.roll(h, -1, axis=0)
        return a_m * h_m + a_s * h + a_p * h_p + bgat

    H1 = gat(P1)
    H2 = gat(P2)

    U = jnp.concatenate([-2.0 * P1, -2.0 * H1], axis=1)
    V = jnp.concatenate([P2, H2], axis=1)
    UU = U * U
    VV = V * V
    quarter = jnp.full((2 * _D, 8), 0.25, dtype=jnp.float32)
    aa_t = jnp.dot(UU, quarter,
                   preferred_element_type=jnp.float32)[:, 0:1]
    ones_row = jnp.full((1, 2 * _D), 1.0, dtype=jnp.float32)

    def mlp_q(p, W1, b1):
        return jnp.maximum(
            jnp.dot(p, W1.T, preferred_element_type=jnp.float32) + b1,
            0.0)

    q1 = mlp_q(P1, Wd1, bd1)
    q2 = mlp_q(P2, Wi1, bi1)
    w2d = jnp.concatenate(
        [jax.lax.transpose(w2d_row, (1, 0)),
         jnp.zeros((_D // 2, 7), jnp.float32)], axis=1)
    r1 = jnp.dot(q1, w2d, preferred_element_type=jnp.float32)[:, 0:1]
    d1 = mean_del + jnp.abs(r1 + bd2)
    r2 = jnp.dot(w2i_row, q2.T,
                 preferred_element_type=jnp.float32)
    d2 = mean_ins + jnp.abs(r2 + bi2)

    for b in range(_B):
        lo, hi = b * _N, (b + 1) * _N
        g2 = jnp.dot(U[lo:hi, :], V[lo:hi, :].T,
                     preferred_element_type=jnp.float32)
        bb = jnp.dot(ones_row, VV[lo:hi, :].T,
                     preferred_element_type=jnp.float32)
        y = (g2 + aa_t[lo:hi, :]) + bb
        colmin = jnp.maximum(jnp.min(y, axis=0, keepdims=True), 0.0)
        rowmin = jnp.maximum(jnp.min(y, axis=1, keepdims=True), 0.0)
        a_v = jnp.minimum(colmin, 2.0 * d2[:, lo:hi])
        b_v = jnp.minimum(rowmin, 2.0 * d1[lo:hi, :])
        out_ref[b] = (jnp.sum(a_v) + jnp.sum(b_v)) / jnp.float32(4 * _N)


def kernel(dense_t1, dense_t2, t1_mask, t2_mask, W_gat, att_src, att_dst,
           b_gat, Wd1, bd1, Wd2, bd2, Wi1, bi1, Wi2, bi2, emb_del, emb_ins):
    del t1_mask, t2_mask
    z7 = jnp.zeros((7 * _D,), jnp.float32)
    z64 = jnp.zeros((_D // 2,), jnp.float32)
    z116 = jnp.zeros((116,), jnp.float32)
    z5 = jnp.zeros((5 * _D,), jnp.float32)
    small = jnp.concatenate([
        att_src, z7, att_dst, z7, b_gat, z7,
        bd1, z64, z7, bi1, z64, z7,
        Wd2.ravel(), z64, Wi2.ravel(), z64,
        bd2, bi2, emb_del[_N - 5:_N, 0], emb_ins[_N - 5:_N, 0], z116, z5,
    ]).reshape(48, _D)
    packed = jnp.concatenate([W_gat, Wd1, Wi1, small], axis=0)

    def fixed(shape):
        return pl.BlockSpec(shape, lambda i: (0, 0))

    out = pl.pallas_call(
        _soft_hd_kernel,
        grid=(1,),
        in_specs=[fixed((_B * _N, _D)), fixed((_B * _N, _D)),
                  fixed((304, _D))],
        out_specs=pl.BlockSpec(memory_space=pltpu.SMEM),
        out_shape=jax.ShapeDtypeStruct((_B,), jnp.float32),
    )(dense_t1.reshape(_B * _N, _D), dense_t2.reshape(_B * _N, _D), packed)
    return out


# speedup vs baseline: 4.0949x; 1.0042x over previous
import jax
import jax.numpy as jnp
from jax.experimental import pallas as pl
from jax.experimental.pallas import tpu as pltpu

_B = 2
_N = 512
_D = 128


def _soft_hd_kernel(p1_ref, p2_ref, wp_ref, out_ref):
    _W0 = 0
    P1 = p1_ref[...]
    P2 = p2_ref[...]
    W = wp_ref[_W0:_W0 + _D, :]
    Wd1 = wp_ref[_W0 + _D:_W0 + _D + 64, :]
    Wi1 = wp_ref[_W0 + _D + 64:_W0 + _D + 128, :]
    asrc = wp_ref[_W0 + 256:_W0 + 257, :]
    adst = wp_ref[_W0 + 264:_W0 + 265, :]
    bgat = wp_ref[_W0 + 272:_W0 + 273, :]
    bd1 = wp_ref[_W0 + 280:_W0 + 281, 0:64]
    bi1 = wp_ref[_W0 + 288:_W0 + 289, 0:64]
    w2d_row = wp_ref[_W0 + 296:_W0 + 297, 0:64]
    w2i_row = wp_ref[_W0 + 297:_W0 + 298, 0:64]
    srow = wp_ref[_W0 + 298:_W0 + 299, :]
    lane = jax.lax.broadcasted_iota(jnp.int32, (1, _D), 1)
    bd2 = jnp.sum(jnp.where(lane == 0, srow, 0.0))
    bi2 = jnp.sum(jnp.where(lane == 1, srow, 0.0))
    mean_del = jnp.sum(
        jnp.where((lane >= 2) & (lane < 7), srow, 0.0)) / 5.0
    mean_ins = jnp.sum(
        jnp.where((lane >= 7) & (lane < 12), srow, 0.0)) / 5.0

    M = _B * _N
    idx = jax.lax.broadcasted_iota(jnp.int32, (M, 1), 0)
    pos = jax.lax.bitwise_and(idx, _N - 1)
    has_m = (pos >= 1).astype(jnp.float32)
    has_p = (pos <= _N - 2).astype(jnp.float32)
    neg_big = jnp.float32(-1e30)

    att = jnp.concatenate(
        [jax.lax.transpose(asrc, (1, 0)),
         jax.lax.transpose(adst, (1, 0))], axis=1)

    def lrelu(x):
        return jnp.where(x >= 0, x, 0.2 * x)

    def gat(p):
        h = jnp.dot(p, W.T, preferred_element_type=jnp.float32)
        e = jnp.dot(h, att, preferred_element_type=jnp.float32)
        es = e[:, 0:1]
        ed = e[:, 1:2]
        es_m = jnp.roll(es, 1, axis=0)
        es_p = jnp.roll(es, -1, axis=0)
        e_m = jnp.where(has_m > 0, lrelu(es_m + ed), neg_big)
        e_s = lrelu(es + ed)
        e_p = jnp.where(has_p > 0, lrelu(es_p + ed), neg_big)
        m = jnp.maximum(jnp.maximum(e_m, e_p), e_s)
        w_m = jnp.exp(e_m - m) * has_m
        w_s = jnp.exp(e_s - m)
        w_p = jnp.exp(e_p - m) * has_p
        inv_s = 1.0 / (w_m + w_s + w_p)
        a_m = w_m * inv_s
        a_s = w_s * inv_s
        a_p = w_p * inv_s
        h_m = jnp.roll(h, 1, axis=0)
        h_p = jnp.roll(h, -1, axis=0)
        return a_m * h_m + a_s * h + a_p * h_p + bgat

    H1 = gat(P1)
    H2 = gat(P2)

    U = jnp.concatenate([-2.0 * P1, -2.0 * H1], axis=1)
    V = jnp.concatenate([P2, H2], axis=1)
    UU = U * U
    VV = V * V
    quarter = jnp.full((2 * _D, 8), 0.25, dtype=jnp.float32)
    aa_t = jnp.dot(UU, quarter,
                   preferred_element_type=jnp.float32)[:, 0:1]
    ones_row = jnp.full((1, 2 * _D), 1.0, dtype=jnp.float32)

    def mlp_q(p, W1, b1):
        return jnp.maximum(
            jnp.dot(p, W1.T, preferred_element_type=jnp.float32) + b1,
            0.0)

    q1 = mlp_q(P1, Wd1, bd1)
    q2 = mlp_q(P2, Wi1, bi1)
    w2d = jnp.concatenate(
        [jax.lax.transpose(w2d_row, (1, 0)),
         jnp.zeros((_D // 2, 7), jnp.float32)], axis=1)
    r1 = jnp.dot(q1, w2d, preferred_element_type=jnp.float32)[:, 0:1]
    d1 = mean_del + jnp.abs(r1 + bd2)
    r2 = jnp.dot(w2i_row, q2.T,
                 preferred_element_type=jnp.float32)
    d2 = mean_ins + jnp.abs(r2 + bi2)

    for b in range(_B):
        lo, hi = b * _N, (b + 1) * _N
        g2 = jnp.dot(U[lo:hi, :], V[lo:hi, :].T,
                     preferred_element_type=jnp.float32)
        bb = jnp.dot(ones_row, VV[lo:hi, :].T,
                     preferred_element_type=jnp.float32)
        y = (g2 + aa_t[lo:hi, :]) + bb
        colmin = jnp.maximum(jnp.min(y, axis=0, keepdims=True), 0.0)
        rowmin = jnp.maximum(jnp.min(y, axis=1, keepdims=True), 0.0)
        a_v = jnp.minimum(colmin, 2.0 * d2[:, lo:hi])
        b_v = jnp.minimum(rowmin, 2.0 * d1[lo:hi, :])
        out_ref[b] = (jnp.sum(a_v) + jnp.sum(b_v)) / jnp.float32(4 * _N)


def kernel(dense_t1, dense_t2, t1_mask, t2_mask, W_gat, att_src, att_dst,
           b_gat, Wd1, bd1, Wd2, bd2, Wi1, bi1, Wi2, bi2, emb_del, emb_ins):
    del t1_mask, t2_mask
    z7 = jnp.zeros((7 * _D,), jnp.float32)
    z64 = jnp.zeros((_D // 2,), jnp.float32)
    z116 = jnp.zeros((116,), jnp.float32)
    z5 = jnp.zeros((5 * _D,), jnp.float32)
    small = jnp.concatenate([
        att_src, z7, att_dst, z7, b_gat, z7,
        bd1, z64, z7, bi1, z64, z7,
        Wd2.ravel(), z64, Wi2.ravel(), z64,
        bd2, bi2, emb_del[_N - 5:_N, 0], emb_ins[_N - 5:_N, 0], z116, z5,
    ]).reshape(48, _D)
    packed = jnp.concatenate([W_gat, Wd1, Wi1, small], axis=0)

    out = pl.pallas_call(
        _soft_hd_kernel,
        in_specs=[pl.BlockSpec(memory_space=pltpu.VMEM),
                  pl.BlockSpec(memory_space=pltpu.VMEM),
                  pl.BlockSpec(memory_space=pltpu.VMEM)],
        out_specs=pl.BlockSpec(memory_space=pltpu.SMEM),
        out_shape=jax.ShapeDtypeStruct((_B,), jnp.float32),
    )(dense_t1.reshape(_B * _N, _D), dense_t2.reshape(_B * _N, _D), packed)
    return out


# speedup vs baseline: 4.1256x; 1.0075x over previous
import jax
import jax.numpy as jnp
from jax.experimental import pallas as pl
from jax.experimental.pallas import tpu as pltpu

_B = 2
_N = 512
_D = 128


def _soft_hd_kernel(p1_ref, p2_ref, wp_ref, out_ref):
    _W0 = 0
    P1 = jnp.reshape(p1_ref[...], (_B * _N, _D))
    P2 = jnp.reshape(p2_ref[...], (_B * _N, _D))
    W = wp_ref[_W0:_W0 + _D, :]
    Wd1 = wp_ref[_W0 + _D:_W0 + _D + 64, :]
    Wi1 = wp_ref[_W0 + _D + 64:_W0 + _D + 128, :]
    asrc = wp_ref[_W0 + 256:_W0 + 257, :]
    adst = wp_ref[_W0 + 264:_W0 + 265, :]
    bgat = wp_ref[_W0 + 272:_W0 + 273, :]
    bd1 = wp_ref[_W0 + 280:_W0 + 281, 0:64]
    bi1 = wp_ref[_W0 + 288:_W0 + 289, 0:64]
    w2d_row = wp_ref[_W0 + 296:_W0 + 297, 0:64]
    w2i_row = wp_ref[_W0 + 297:_W0 + 298, 0:64]
    srow = wp_ref[_W0 + 298:_W0 + 299, :]
    lane = jax.lax.broadcasted_iota(jnp.int32, (1, _D), 1)
    bd2 = jnp.sum(jnp.where(lane == 0, srow, 0.0))
    bi2 = jnp.sum(jnp.where(lane == 1, srow, 0.0))
    mean_del = jnp.sum(
        jnp.where((lane >= 2) & (lane < 7), srow, 0.0)) / 5.0
    mean_ins = jnp.sum(
        jnp.where((lane >= 7) & (lane < 12), srow, 0.0)) / 5.0

    M = _B * _N
    lpos = jax.lax.bitwise_and(
        jax.lax.broadcasted_iota(jnp.int32, (3, M), 1), _N - 1)
    srow3 = jax.lax.broadcasted_iota(jnp.int32, (3, M), 0)
    invalid3 = ((srow3 == 0) & (lpos == 0)) | ((srow3 == 2) & (lpos == _N - 1))
    valid3f = jnp.where(invalid3, 0.0, 1.0)
    neg_big = jnp.float32(-1e30)

    att2 = jnp.concatenate([asrc, adst], axis=0)

    def lrelu(x):
        return jnp.where(x >= 0, x, 0.2 * x)

    def gat(p):
        h = jnp.dot(p, W.T, preferred_element_type=jnp.float32)
        e = jnp.dot(att2, h.T, preferred_element_type=jnp.float32)
        es = e[0:1, :]
        ed = e[1:2, :]
        pre3 = jnp.concatenate(
            [jnp.roll(es, 1, axis=1), es, jnp.roll(es, -1, axis=1)],
            axis=0)
        e3 = jnp.where(invalid3, neg_big, lrelu(pre3 + ed))
        m = jnp.max(e3, axis=0, keepdims=True)
        w3 = jnp.exp(e3 - m) * valid3f
        a3 = w3 * (1.0 / jnp.sum(w3, axis=0, keepdims=True))
        eye38 = (jax.lax.broadcasted_iota(jnp.int32, (3, 8), 0)
                 == jax.lax.broadcasted_iota(jnp.int32, (3, 8), 1)
                 ).astype(jnp.float32)
        A3 = jnp.dot(a3.T, eye38, preferred_element_type=jnp.float32)
        a_m = A3[:, 0:1]
        a_s = A3[:, 1:2]
        a_p = A3[:, 2:3]
        h_m = jnp.roll(h, 1, axis=0)
        h_p = jnp.roll(h, -1, axis=0)
        return a_m * h_m + a_s * h + a_p * h_p + bgat

    H1 = gat(P1)
    H2 = gat(P2)

    U = jnp.concatenate([-2.0 * P1, -2.0 * H1], axis=1)
    V = jnp.concatenate([P2, H2], axis=1)
    UU = U * U
    VV = V * V
    quarter = jnp.full((2 * _D, 8), 0.25, dtype=jnp.float32)
    aa_t = jnp.dot(UU, quarter,
                   preferred_element_type=jnp.float32)[:, 0:1]
    ones_row = jnp.full((1, 2 * _D), 1.0, dtype=jnp.float32)

    def mlp_q(p, W1, b1):
        return jnp.maximum(
            jnp.dot(p, W1.T, preferred_element_type=jnp.float32) + b1,
            0.0)

    q1 = mlp_q(P1, Wd1, bd1)
    q2 = mlp_q(P2, Wi1, bi1)
    w2d = jnp.concatenate(
        [jax.lax.transpose(w2d_row, (1, 0)),
         jnp.zeros((_D // 2, 7), jnp.float32)], axis=1)
    r1 = jnp.dot(q1, w2d, preferred_element_type=jnp.float32)[:, 0:1]
    d1 = mean_del + jnp.abs(r1 + bd2)
    r2 = jnp.dot(w2i_row, q2.T,
                 preferred_element_type=jnp.float32)
    d2 = mean_ins + jnp.abs(r2 + bi2)

    for b in range(_B):
        lo, hi = b * _N, (b + 1) * _N
        g2 = jnp.dot(U[lo:hi, :], V[lo:hi, :].T,
                     preferred_element_type=jnp.float32)
        bb = jnp.dot(ones_row, VV[lo:hi, :].T,
                     preferred_element_type=jnp.float32)
        y = (g2 + aa_t[lo:hi, :]) + bb
        colmin = jnp.maximum(jnp.min(y, axis=0, keepdims=True), 0.0)
        rowmin = jnp.maximum(jnp.min(y, axis=1, keepdims=True), 0.0)
        a_v = jnp.minimum(colmin, 2.0 * d2[:, lo:hi])
        b_v = jnp.minimum(rowmin, 2.0 * d1[lo:hi, :])
        out_ref[b] = (jnp.sum(a_v) + jnp.sum(b_v)) / jnp.float32(4 * _N)


def kernel(dense_t1, dense_t2, t1_mask, t2_mask, W_gat, att_src, att_dst,
           b_gat, Wd1, bd1, Wd2, bd2, Wi1, bi1, Wi2, bi2, emb_del, emb_ins):
    del t1_mask, t2_mask
    z7 = jnp.zeros((7 * _D,), jnp.float32)
    z64 = jnp.zeros((_D // 2,), jnp.float32)
    z116 = jnp.zeros((116,), jnp.float32)
    z5 = jnp.zeros((5 * _D,), jnp.float32)
    small = jnp.concatenate([
        att_src, z7, att_dst, z7, b_gat, z7,
        bd1, z64, z7, bi1, z64, z7,
        Wd2.ravel(), z64, Wi2.ravel(), z64,
        bd2, bi2, emb_del[_N - 5:_N, 0], emb_ins[_N - 5:_N, 0], z116, z5,
    ]).reshape(48, _D)
    packed = jnp.concatenate([W_gat, Wd1, Wi1, small], axis=0)

    out = pl.pallas_call(
        _soft_hd_kernel,
        in_specs=[pl.BlockSpec(memory_space=pltpu.VMEM),
                  pl.BlockSpec(memory_space=pltpu.VMEM),
                  pl.BlockSpec(memory_space=pltpu.VMEM)],
        out_specs=pl.BlockSpec(memory_space=pltpu.SMEM),
        out_shape=jax.ShapeDtypeStruct((_B,), jnp.float32),
    )(dense_t1, dense_t2, packed)
    return out


# speedup vs baseline: 4.2244x; 1.0239x over previous
import jax
import jax.numpy as jnp
from jax.experimental import pallas as pl
from jax.experimental.pallas import tpu as pltpu

_B = 2
_N = 512
_D = 128


def _soft_hd_kernel(p1_ref, p2_ref, wp_ref, out_ref):
    _W0 = 0
    P1 = jnp.reshape(p1_ref[...], (_B * _N, _D))
    P2 = jnp.reshape(p2_ref[...], (_B * _N, _D))
    W = wp_ref[_W0:_W0 + _D, :]
    Wd1 = wp_ref[_W0 + _D:_W0 + _D + 64, :]
    Wi1 = wp_ref[_W0 + _D + 64:_W0 + _D + 128, :]
    asrc = wp_ref[_W0 + 256:_W0 + 257, :]
    adst = wp_ref[_W0 + 264:_W0 + 265, :]
    bgat = wp_ref[_W0 + 272:_W0 + 273, :]
    bd1 = wp_ref[_W0 + 280:_W0 + 281, 0:64]
    bi1 = wp_ref[_W0 + 288:_W0 + 289, 0:64]
    w2d_row = wp_ref[_W0 + 296:_W0 + 297, 0:64]
    w2i_row = wp_ref[_W0 + 297:_W0 + 298, 0:64]
    srow = wp_ref[_W0 + 298:_W0 + 299, :]
    lane = jax.lax.broadcasted_iota(jnp.int32, (1, _D), 1)
    bd2 = jnp.sum(jnp.where(lane == 0, srow, 0.0))
    bi2 = jnp.sum(jnp.where(lane == 1, srow, 0.0))
    mean_del = jnp.sum(
        jnp.where((lane >= 2) & (lane < 7), srow, 0.0)) / 5.0
    mean_ins = jnp.sum(
        jnp.where((lane >= 7) & (lane < 12), srow, 0.0)) / 5.0

    M = _B * _N
    M2 = 2 * M
    lpos = jax.lax.bitwise_and(
        jax.lax.broadcasted_iota(jnp.int32, (3, M2), 1), _N - 1)
    srow3 = jax.lax.broadcasted_iota(jnp.int32, (3, M2), 0)
    invalid3 = ((srow3 == 0) & (lpos == 0)) | ((srow3 == 2) & (lpos == _N - 1))
    valid3f = jnp.where(invalid3, 0.0, 1.0)
    neg_big = jnp.float32(-1e30)

    att2 = jnp.concatenate([asrc, adst], axis=0)

    def lrelu(x):
        return jnp.where(x >= 0, x, 0.2 * x)

    P12 = jnp.concatenate([P1, P2], axis=0)
    h = jnp.dot(P12, W.T, preferred_element_type=jnp.float32)
    e = jnp.dot(att2, h.T, preferred_element_type=jnp.float32)
    es = e[0:1, :]
    ed = e[1:2, :]
    pre3 = jnp.concatenate(
        [jnp.roll(es, 1, axis=1), es, jnp.roll(es, -1, axis=1)],
        axis=0)
    e3 = jnp.where(invalid3, neg_big, lrelu(pre3 + ed))
    m = jnp.max(e3, axis=0, keepdims=True)
    w3 = jnp.exp(e3 - m) * valid3f
    a3 = w3 * (1.0 / jnp.sum(w3, axis=0, keepdims=True))
    eye38 = (jax.lax.broadcasted_iota(jnp.int32, (3, 8), 0)
             == jax.lax.broadcasted_iota(jnp.int32, (3, 8), 1)
             ).astype(jnp.float32)
    A3 = jnp.dot(a3.T, eye38, preferred_element_type=jnp.float32)
    a_m = A3[:, 0:1]
    a_s = A3[:, 1:2]
    a_p = A3[:, 2:3]
    h_m = jnp.roll(h, 1, axis=0)
    h_p = jnp.roll(h, -1, axis=0)
    H12 = a_m * h_m + a_s * h + a_p * h_p + bgat
    H1 = H12[0:M, :]
    H2 = H12[M:M2, :]

    U = jnp.concatenate([-2.0 * P1, -2.0 * H1], axis=1)
    V = jnp.concatenate([P2, H2], axis=1)
    UU = U * U
    VV = V * V
    quarter = jnp.full((2 * _D, 8), 0.25, dtype=jnp.float32)
    aa_t = jnp.dot(UU, quarter,
                   preferred_element_type=jnp.float32)[:, 0:1]
    ones_row = jnp.full((1, 2 * _D), 1.0, dtype=jnp.float32)

    def mlp_q(p, W1, b1):
        return jnp.maximum(
            jnp.dot(p, W1.T, preferred_element_type=jnp.float32) + b1,
            0.0)

    q1 = mlp_q(P1, Wd1, bd1)
    q2 = mlp_q(P2, Wi1, bi1)
    w2d = jnp.concatenate(
        [jax.lax.transpose(w2d_row, (1, 0)),
         jnp.zeros((_D // 2, 7), jnp.float32)], axis=1)
    r1 = jnp.dot(q1, w2d, preferred_element_type=jnp.float32)[:, 0:1]
    d1 = mean_del + jnp.abs(r1 + bd2)
    r2 = jnp.dot(w2i_row, q2.T,
                 preferred_element_type=jnp.float32)
    d2 = mean_ins + jnp.abs(r2 + bi2)

    bb_full = jnp.dot(ones_row, VV.T,
                      preferred_element_type=jnp.float32)
    for b in range(_B):
        lo, hi = b * _N, (b + 1) * _N
        g2 = jnp.dot(U[lo:hi, :], V[lo:hi, :].T,
                     preferred_element_type=jnp.float32)
        y = (g2 + aa_t[lo:hi, :]) + bb_full[:, lo:hi]
        colmin = jnp.maximum(jnp.min(y, axis=0, keepdims=True), 0.0)
        rowmin = jnp.maximum(jnp.min(y, axis=1, keepdims=True), 0.0)
        a_v = jnp.minimum(colmin, 2.0 * d2[:, lo:hi])
        b_v = jnp.minimum(rowmin, 2.0 * d1[lo:hi, :])
        out_ref[b] = (jnp.sum(a_v) + jnp.sum(b_v)) / jnp.float32(4 * _N)


def kernel(dense_t1, dense_t2, t1_mask, t2_mask, W_gat, att_src, att_dst,
           b_gat, Wd1, bd1, Wd2, bd2, Wi1, bi1, Wi2, bi2, emb_del, emb_ins):
    del t1_mask, t2_mask
    z7 = jnp.zeros((7 * _D,), jnp.float32)
    z64 = jnp.zeros((_D // 2,), jnp.float32)
    z116 = jnp.zeros((116,), jnp.float32)
    z5 = jnp.zeros((5 * _D,), jnp.float32)
    small = jnp.concatenate([
        att_src, z7, att_dst, z7, b_gat, z7,
        bd1, z64, z7, bi1, z64, z7,
        Wd2.ravel(), z64, Wi2.ravel(), z64,
        bd2, bi2, emb_del[_N - 5:_N, 0], emb_ins[_N - 5:_N, 0], z116, z5,
    ]).reshape(48, _D)
    packed = jnp.concatenate([W_gat, Wd1, Wi1, small], axis=0)

    out = pl.pallas_call(
        _soft_hd_kernel,
        in_specs=[pl.BlockSpec(memory_space=pltpu.VMEM),
                  pl.BlockSpec(memory_space=pltpu.VMEM),
                  pl.BlockSpec(memory_space=pltpu.VMEM)],
        out_specs=pl.BlockSpec(memory_space=pltpu.SMEM),
        out_shape=jax.ShapeDtypeStruct((_B,), jnp.float32),
    )(dense_t1, dense_t2, packed)
    return out
